# Initial kernel scaffold; baseline (speedup 1.0000x reference)
import jax
import jax.numpy as jnp
from jax.experimental import pallas as pl


def kernel(x, adj_t, edge_attr, W_m1, b_m1, W_m2, b_m2, W_me, b_me, W_net, b_net, W_o1, b_o1, W_o2, b_o2):
    raise NotImplementedError("write your pallas kernel here")


# speedup vs baseline: 1.3498x; 1.3498x over previous
import functools

import jax
import jax.numpy as jnp
from jax import lax
from jax.experimental import pallas as pl
from jax.experimental.pallas import tpu as pltpu
import jax.experimental.pallas.tpu_sc as plsc

N = 10000
E = 320000
D = 128
DE = 16

NW = 32
EPW = 10240
E_PAD = NW * EPW
K = 128
NPT = 320
N_PAD = NW * NPT

BE = 1024
BN = 1000


def _node_mm2_body(x_ref, w1_ref, b1_ref, w2_ref, b2_ref, o1_ref, o2_ref):
    xb = x_ref[...]
    o1_ref[...] = jnp.dot(xb, w1_ref[...], preferred_element_type=jnp.float32) + b1_ref[...]
    o2_ref[...] = jnp.dot(xb, w2_ref[...], preferred_element_type=jnp.float32) + b2_ref[...]


def _node_mm2(x, w1, b1, w2, b2):
    grid = N // BN
    return pl.pallas_call(
        _node_mm2_body,
        grid=(grid,),
        in_specs=[
            pl.BlockSpec((BN, D), lambda i: (i, 0)),
            pl.BlockSpec((D, D), lambda i: (0, 0)),
            pl.BlockSpec((D,), lambda i: (0,)),
            pl.BlockSpec((D, D), lambda i: (0, 0)),
            pl.BlockSpec((D,), lambda i: (0,)),
        ],
        out_specs=[
            pl.BlockSpec((BN, D), lambda i: (i, 0)),
            pl.BlockSpec((BN, D), lambda i: (i, 0)),
        ],
        out_shape=[
            jax.ShapeDtypeStruct((N, D), jnp.float32),
            jax.ShapeDtypeStruct((N, D), jnp.float32),
        ],
    )(x, w1, b1, w2, b2)


def _gather_body(m1_hbm, m2_hbm, row_hbm, col_hbm, g1_hbm, g2_hbm,
                 ridx, cidx, buf1, buf2, sem1, sem2):
    wid = lax.axis_index("s") * 2 + lax.axis_index("c")
    base = wid * EPW

    def chunk(ci, carry):
        off = base + ci * K
        pltpu.sync_copy(row_hbm.at[pl.ds(off, K)], ridx)
        pltpu.sync_copy(col_hbm.at[pl.ds(off, K)], cidx)
        cp1 = pltpu.async_copy(m1_hbm.at[ridx], buf1, sem1)
        cp2 = pltpu.async_copy(m2_hbm.at[cidx], buf2, sem2)
        cp1.wait()
        cp2.wait()
        pltpu.sync_copy(buf1, g1_hbm.at[pl.ds(off, K)])
        pltpu.sync_copy(buf2, g2_hbm.at[pl.ds(off, K)])
        return carry

    lax.fori_loop(0, EPW // K, chunk, 0)


def _edge_gather(m1, m2, row_pad, col_pad):
    f = functools.partial(
        pl.kernel,
        out_type=[
            jax.ShapeDtypeStruct((E_PAD, D), jnp.float32),
            jax.ShapeDtypeStruct((E_PAD, D), jnp.float32),
        ],
        mesh=plsc.VectorSubcoreMesh(core_axis_name="c", subcore_axis_name="s"),
        scratch_types=[
            pltpu.VMEM((K,), jnp.int32),
            pltpu.VMEM((K,), jnp.int32),
            pltpu.VMEM((K, D), jnp.float32),
            pltpu.VMEM((K, D), jnp.float32),
            pltpu.SemaphoreType.DMA,
            pltpu.SemaphoreType.DMA,
        ],
    )(_gather_body)
    return f(m1, m2, row_pad, col_pad)


def _edge_mlp_body(g1_ref, g2_ref, ea_ref, wme_ref, bme_ref, wnet_ref, bnet_ref, h_ref):
    msge = jnp.dot(ea_ref[...], wme_ref[...], preferred_element_type=jnp.float32)
    msg = jnp.maximum(g1_ref[...] + g2_ref[...] + msge + bme_ref[...], 0.0)
    h = jnp.dot(msg, wnet_ref[...], preferred_element_type=jnp.float32) + bnet_ref[...]
    h_ref[...] = jnp.maximum(h, 0.0)


def _edge_mlp(g1, g2, ea_pad, wme, bme, wnet, bnet):
    grid = E_PAD // BE
    return pl.pallas_call(
        _edge_mlp_body,
        grid=(grid,),
        in_specs=[
            pl.BlockSpec((BE, D), lambda i: (i, 0)),
            pl.BlockSpec((BE, D), lambda i: (i, 0)),
            pl.BlockSpec((BE, DE), lambda i: (i, 0)),
            pl.BlockSpec((DE, D), lambda i: (0, 0)),
            pl.BlockSpec((D,), lambda i: (0,)),
            pl.BlockSpec((D, D), lambda i: (0, 0)),
            pl.BlockSpec((D,), lambda i: (0,)),
        ],
        out_specs=pl.BlockSpec((BE, D), lambda i: (i, 0)),
        out_shape=jax.ShapeDtypeStruct((E_PAD, D), jnp.float32),
    )(g1, g2, ea_pad, wme, bme, wnet, bnet)


def _segmax_body(h_hbm, row_hbm, ptr_hbm, agg_hbm, ptr_v, rv, hbuf, acc):
    wid = lax.axis_index("s") * 2 + lax.axis_index("c")
    n0 = wid * NPT

    pltpu.sync_copy(ptr_hbm, ptr_v)

    pv = ptr_v[pl.ds(wid, 16)]
    p0 = pv[0]
    p1 = pv[1]

    zeros = jnp.zeros((16,), jnp.float32)

    def zrow(r, carry):
        for j in range(D // 16):
            acc[r, pl.ds(j * 16, 16)] = zeros
        return carry

    lax.fori_loop(0, NPT, zrow, 0)

    s_start = (p0 // 8) * 8
    nchunks = (p1 - s_start + K - 1) // K

    def chunk(ci, carry):
        s = s_start + ci * K
        pltpu.sync_copy(row_hbm.at[pl.ds(s, K)], rv.at[pl.ds(0, K)])
        pltpu.sync_copy(h_hbm.at[pl.ds(s, K)], hbuf)
        r_lo = jnp.maximum(0, p0 - s)
        r_hi = jnp.minimum(K, p1 - s)

        def edge(r, c2):
            rid = rv[pl.ds(r, 16)][0]
            loc = rid - n0
            for j in range(D // 16):
                sl = pl.ds(j * 16, 16)
                acc[loc, sl] = jnp.maximum(acc[loc, sl], hbuf[r, sl])
            return c2

        lax.fori_loop(r_lo, r_hi, edge, 0)
        return carry

    lax.fori_loop(0, nchunks, chunk, 0)
    pltpu.sync_copy(acc, agg_hbm.at[pl.ds(n0, NPT)])


def _segmax(h, row_pad, ptr):
    f = functools.partial(
        pl.kernel,
        out_type=jax.ShapeDtypeStruct((N_PAD, D), jnp.float32),
        mesh=plsc.VectorSubcoreMesh(core_axis_name="c", subcore_axis_name="s"),
        scratch_types=[
            pltpu.VMEM((48,), jnp.int32),
            pltpu.VMEM((K + 16,), jnp.int32),
            pltpu.VMEM((K, D), jnp.float32),
            pltpu.VMEM((NPT, D), jnp.float32),
        ],
    )(_segmax_body)
    return f(h, row_pad, ptr)


def _out_body(x_ref, agg_ref, wo1_ref, bo1_ref, wo2_ref, bo2_ref, o_ref):
    h1 = jnp.dot(x_ref[...], wo1_ref[...], preferred_element_type=jnp.float32) + bo1_ref[...]
    h2 = jnp.dot(agg_ref[...], wo2_ref[...], preferred_element_type=jnp.float32) + bo2_ref[...]
    o_ref[...] = h1 + h2


def _out_combine(x, agg, wo1, bo1, wo2, bo2):
    grid = N // BN
    return pl.pallas_call(
        _out_body,
        grid=(grid,),
        in_specs=[
            pl.BlockSpec((BN, D), lambda i: (i, 0)),
            pl.BlockSpec((BN, D), lambda i: (i, 0)),
            pl.BlockSpec((D, D), lambda i: (0, 0)),
            pl.BlockSpec((D,), lambda i: (0,)),
            pl.BlockSpec((D, D), lambda i: (0, 0)),
            pl.BlockSpec((D,), lambda i: (0,)),
        ],
        out_specs=pl.BlockSpec((BN, D), lambda i: (i, 0)),
        out_shape=jax.ShapeDtypeStruct((N, D), jnp.float32),
    )(x, agg, wo1, bo1, wo2, bo2)


def kernel(x, adj_t, edge_attr, W_m1, b_m1, W_m2, b_m2, W_me, b_me,
           W_net, b_net, W_o1, b_o1, W_o2, b_o2):
    row = adj_t[0].astype(jnp.int32)
    col = adj_t[1].astype(jnp.int32)

    row_pad = jnp.pad(row, (0, E_PAD - E))
    col_pad = jnp.pad(col, (0, E_PAD - E))
    ea_pad = jnp.pad(edge_attr, ((0, E_PAD - E), (0, 0)))

    bounds = jnp.arange(NW + 1, dtype=jnp.int32) * NPT
    ptr = jnp.searchsorted(row, bounds, side="left").astype(jnp.int32)
    ptr = jnp.pad(ptr, (0, 48 - (NW + 1)))

    msg1, msg2 = _node_mm2(x, W_m1, b_m1, W_m2, b_m2)
    g1, g2 = _edge_gather(msg1, msg2, row_pad, col_pad)
    h = _edge_mlp(g1, g2, ea_pad, W_me, b_me, W_net, b_net)
    agg = _segmax(h, row_pad, ptr)
    out = _out_combine(x, agg[:N], W_o1, b_o1, W_o2, b_o2)
    return out
